# Initial kernel scaffold; baseline (speedup 1.0000x reference)
import jax
import jax.numpy as jnp
from jax.experimental import pallas as pl


def kernel(nodes, edges, senders, receivers, globals_, n_node, n_edge, W_edge, b_edge, W_node, b_node, W_glob, b_glob):
    raise NotImplementedError("write your pallas kernel here")


# speedup vs baseline: 4.0574x; 4.0574x over previous
import functools

import jax
import jax.numpy as jnp
from jax import lax
from jax.experimental import pallas as pl
from jax.experimental.pallas import tpu as pltpu
from jax.experimental.pallas import tpu_sc as plsc

F32 = jnp.float32


def _proj_kernel(nodes_ref, ws_ref, wr_ref, ps_ref, pr_ref):
    x = nodes_ref[...]
    ps_ref[...] = jnp.dot(x, ws_ref[...], preferred_element_type=F32)
    pr_ref[...] = jnp.dot(x, wr_ref[...], preferred_element_type=F32)


def _edge_pre_kernel(edges_ref, we_ref, g_ref, wg_ref, b_ref, q_ref):
    ce = jnp.dot(g_ref[...], wg_ref[...], preferred_element_type=F32) + b_ref[...]
    q_ref[...] = jnp.dot(edges_ref[...], we_ref[...], preferred_element_type=F32) + ce


def _node_glob_kernel(nodes_ref, sp_ref, rp_ref, g_ref, wnn_ref, wns_ref,
                      wnr_ref, wng_ref, bn_ref, wgn_ref, wge_ref, wgg_ref,
                      bg_ref, nn_ref, gout_ref, nsum_acc, esum_acc):
    i = pl.program_id(0)
    s_agg = sp_ref[0] + sp_ref[1]
    r_agg = rp_ref[0] + rp_ref[1]
    cn = jnp.dot(g_ref[...], wng_ref[...], preferred_element_type=F32) + bn_ref[...]
    x = (jnp.dot(nodes_ref[...], wnn_ref[...], preferred_element_type=F32)
         + jnp.dot(s_agg, wns_ref[...], preferred_element_type=F32)
         + jnp.dot(r_agg, wnr_ref[...], preferred_element_type=F32)
         + cn)
    nn = jnp.maximum(x, 0.0)
    nn_ref[...] = nn

    @pl.when(i == 0)
    def _():
        nsum_acc[...] = jnp.zeros_like(nsum_acc)
        esum_acc[...] = jnp.zeros_like(esum_acc)

    nsum_acc[...] += jnp.sum(nn, axis=0, keepdims=True)
    esum_acc[...] += jnp.sum(s_agg, axis=0, keepdims=True)

    @pl.when(i == pl.num_programs(0) - 1)
    def _():
        gi = (jnp.dot(nsum_acc[...], wgn_ref[...], preferred_element_type=F32)
              + jnp.dot(esum_acc[...], wge_ref[...], preferred_element_type=F32)
              + jnp.dot(g_ref[...], wgg_ref[...], preferred_element_type=F32)
              + bg_ref[...])
        gout_ref[...] = jnp.maximum(gi, 0.0)


_C = 128
_NSUB = 16
_NCORE = 2


def _pad_nodes(n):
    step = 8 * _NSUB
    return ((n + step - 1) // step) * step


def _zero_vmem_rows(buf, nrows, ncols):
    def row(i, _):
        for j in range(ncols // 16):
            buf[i, pl.ds(j * 16, 16)] = jnp.zeros((16,), F32)
        return _
    lax.fori_loop(0, nrows, row, None)


def _tile_chunks(wid, total_chunks):
    nbase = total_chunks // (_NCORE * _NSUB)
    rem = total_chunks - nbase * _NCORE * _NSUB
    extra = jnp.where(wid < rem, 1, 0)
    start = wid * nbase + jnp.minimum(wid, rem)
    return start, nbase + extra


def _edge_sc_body(ps_hbm, pr_hbm, q_hbm, s_hbm, r_hbm, en_hbm, sp_hbm,
                  idx_s, idx_r, qbuf, psbuf, prbuf, agg, sem1, sem2):
    cid = lax.axis_index("c")
    sid = lax.axis_index("s")
    wid = cid * _NSUB + sid
    n_nodes = agg.shape[0]
    rows_per_tile = n_nodes // _NSUB
    zb = sid * rows_per_tile

    _zero_vmem_rows(qbuf, _C, 128)
    nfull = rows_per_tile // _C
    tail = rows_per_tile - nfull * _C
    for t in range(nfull):
        pltpu.sync_copy(qbuf, agg.at[pl.ds(zb + t * _C, _C)])
    if tail:
        pltpu.sync_copy(qbuf.at[pl.ds(0, tail)], agg.at[pl.ds(zb + nfull * _C, tail)])
    plsc.subcore_barrier()

    total_chunks = q_hbm.shape[0] // _C
    start, nch = _tile_chunks(wid, total_chunks)

    def chunk(g, _):
        base = (start + g) * _C
        pltpu.sync_copy(s_hbm.at[pl.ds(base, _C)], idx_s)
        pltpu.sync_copy(r_hbm.at[pl.ds(base, _C)], idx_r)
        pltpu.sync_copy(q_hbm.at[pl.ds(base, _C)], qbuf)
        cp1 = pltpu.async_copy(ps_hbm.at[idx_s], psbuf, sem1)
        cp2 = pltpu.async_copy(pr_hbm.at[idx_r], prbuf, sem2)
        cp1.wait()
        cp2.wait()

        def row(i, _):
            for j in range(8):
                sl = pl.ds(j * 16, 16)
                qbuf[i, sl] = jnp.maximum(
                    qbuf[i, sl] + psbuf[i, sl] + prbuf[i, sl], 0.0)
            return _

        lax.fori_loop(0, _C, row, None)
        pltpu.sync_copy(qbuf, en_hbm.at[pl.ds(base, _C)])
        pltpu.sync_copy(qbuf, agg.at[idx_s], add=True)
        return _

    lax.fori_loop(0, nch, chunk, None)
    plsc.subcore_barrier()
    pltpu.sync_copy(agg.at[pl.ds(zb, rows_per_tile)],
                    sp_hbm.at[cid, pl.ds(zb, rows_per_tile)])


def _recv_sc_body(en_hbm, r_hbm, rp_hbm, idx_r, buf, agg, sem1):
    cid = lax.axis_index("c")
    sid = lax.axis_index("s")
    wid = cid * _NSUB + sid
    n_nodes = agg.shape[0]
    rows_per_tile = n_nodes // _NSUB
    zb = sid * rows_per_tile

    _zero_vmem_rows(buf, _C, 128)
    nfull = rows_per_tile // _C
    tail = rows_per_tile - nfull * _C
    for t in range(nfull):
        pltpu.sync_copy(buf, agg.at[pl.ds(zb + t * _C, _C)])
    if tail:
        pltpu.sync_copy(buf.at[pl.ds(0, tail)], agg.at[pl.ds(zb + nfull * _C, tail)])
    plsc.subcore_barrier()

    total_chunks = en_hbm.shape[0] // _C
    start, nch = _tile_chunks(wid, total_chunks)

    def chunk(g, _):
        base = (start + g) * _C
        pltpu.sync_copy(r_hbm.at[pl.ds(base, _C)], idx_r)
        pltpu.sync_copy(en_hbm.at[pl.ds(base, _C)], buf)
        pltpu.sync_copy(buf, agg.at[idx_r], add=True)
        return _

    lax.fori_loop(0, nch, chunk, None)
    plsc.subcore_barrier()
    pltpu.sync_copy(agg.at[pl.ds(zb, rows_per_tile)],
                    rp_hbm.at[cid, pl.ds(zb, rows_per_tile)])


def kernel(nodes, edges, senders, receivers, globals_, n_node, n_edge,
           W_edge, b_edge, W_node, b_node, W_glob, b_glob):
    N, F = nodes.shape
    E, DE = edges.shape
    DG = globals_.shape[1]
    DEO = b_edge.shape[0]
    DNO = b_node.shape[0]

    senders = senders.astype(jnp.int32)
    receivers = receivers.astype(jnp.int32)

    we_e = W_edge[:DE]
    we_s = W_edge[DE:DE + F]
    we_r = W_edge[DE + F:DE + 2 * F]
    we_g = W_edge[DE + 2 * F:]
    wn_n = W_node[:F]
    wn_s = W_node[F:F + DEO]
    wn_r = W_node[F + DEO:F + 2 * DEO]
    wn_g = W_node[F + 2 * DEO:]
    wg_n = W_glob[:DNO]
    wg_e = W_glob[DNO:DNO + DEO]
    wg_g = W_glob[DNO + DEO:]
    b_edge2 = b_edge.reshape(1, DEO)
    b_node2 = b_node.reshape(1, DNO)
    b_glob2 = b_glob.reshape(1, -1)

    ps, pr = pl.pallas_call(
        _proj_kernel,
        out_shape=(jax.ShapeDtypeStruct((N, DEO), F32),
                   jax.ShapeDtypeStruct((N, DEO), F32)),
    )(nodes, we_s, we_r)

    BE = 8000
    q = pl.pallas_call(
        _edge_pre_kernel,
        grid=(E // BE,),
        in_specs=[
            pl.BlockSpec((BE, DE), lambda i: (i, 0)),
            pl.BlockSpec((DE, DEO), lambda i: (0, 0)),
            pl.BlockSpec((1, DG), lambda i: (0, 0)),
            pl.BlockSpec((DG, DEO), lambda i: (0, 0)),
            pl.BlockSpec((1, DEO), lambda i: (0, 0)),
        ],
        out_specs=pl.BlockSpec((BE, DEO), lambda i: (i, 0)),
        out_shape=jax.ShapeDtypeStruct((E, DEO), F32),
    )(edges, we_e, globals_, we_g, b_edge2)

    NP = _pad_nodes(N)
    mesh = plsc.VectorSubcoreMesh(core_axis_name="c", subcore_axis_name="s")
    edges_new, sent_part = pl.kernel(
        _edge_sc_body,
        out_type=(jax.ShapeDtypeStruct((E, DEO), F32),
                  jax.ShapeDtypeStruct((_NCORE, NP, DEO), F32)),
        mesh=mesh,
        scratch_types=(
            pltpu.VMEM((_C,), jnp.int32),
            pltpu.VMEM((_C,), jnp.int32),
            pltpu.VMEM((_C, DEO), F32),
            pltpu.VMEM((_C, DEO), F32),
            pltpu.VMEM((_C, DEO), F32),
            pltpu.VMEM_SHARED((NP, DEO), F32),
            pltpu.SemaphoreType.DMA,
            pltpu.SemaphoreType.DMA,
        ),
    )(ps, pr, q, senders, receivers)

    recv_part = pl.kernel(
        _recv_sc_body,
        out_type=jax.ShapeDtypeStruct((_NCORE, NP, DEO), F32),
        mesh=mesh,
        scratch_types=(
            pltpu.VMEM((_C,), jnp.int32),
            pltpu.VMEM((_C, DEO), F32),
            pltpu.VMEM_SHARED((NP, DEO), F32),
            pltpu.SemaphoreType.DMA,
        ),
    )(edges_new, receivers)

    BN = 1000
    nodes_new, globals_new = pl.pallas_call(
        _node_glob_kernel,
        grid=(N // BN,),
        in_specs=[
            pl.BlockSpec((BN, F), lambda i: (i, 0)),
            pl.BlockSpec((_NCORE, BN, DEO), lambda i: (0, i, 0)),
            pl.BlockSpec((_NCORE, BN, DEO), lambda i: (0, i, 0)),
            pl.BlockSpec((1, DG), lambda i: (0, 0)),
            pl.BlockSpec((F, DNO), lambda i: (0, 0)),
            pl.BlockSpec((DEO, DNO), lambda i: (0, 0)),
            pl.BlockSpec((DEO, DNO), lambda i: (0, 0)),
            pl.BlockSpec((DG, DNO), lambda i: (0, 0)),
            pl.BlockSpec((1, DNO), lambda i: (0, 0)),
            pl.BlockSpec((DNO, b_glob.shape[0]), lambda i: (0, 0)),
            pl.BlockSpec((DEO, b_glob.shape[0]), lambda i: (0, 0)),
            pl.BlockSpec((DG, b_glob.shape[0]), lambda i: (0, 0)),
            pl.BlockSpec((1, b_glob.shape[0]), lambda i: (0, 0)),
        ],
        out_specs=(pl.BlockSpec((BN, DNO), lambda i: (i, 0)),
                   pl.BlockSpec((1, b_glob.shape[0]), lambda i: (0, 0))),
        out_shape=(jax.ShapeDtypeStruct((N, DNO), F32),
                   jax.ShapeDtypeStruct((1, b_glob.shape[0]), F32)),
        scratch_shapes=[pltpu.VMEM((1, DNO), F32),
                        pltpu.VMEM((1, DEO), F32)],
    )(nodes, sent_part, recv_part, globals_, wn_n, wn_s, wn_r, wn_g,
      b_node2, wg_n, wg_e, wg_g, b_glob2)

    return nodes_new, edges_new, globals_new


# speedup vs baseline: 6.0986x; 1.5031x over previous
import functools

import jax
import jax.numpy as jnp
from jax import lax
from jax.experimental import pallas as pl
from jax.experimental.pallas import tpu as pltpu
from jax.experimental.pallas import tpu_sc as plsc

F32 = jnp.float32


def _proj_kernel(nodes_ref, ws_ref, wr_ref, ps_ref, pr_ref):
    x = nodes_ref[...]
    ps_ref[...] = jnp.dot(x, ws_ref[...], preferred_element_type=F32)
    pr_ref[...] = jnp.dot(x, wr_ref[...], preferred_element_type=F32)


def _edge_pre_kernel(edges_ref, we_ref, g_ref, wg_ref, b_ref, q_ref):
    ce = jnp.dot(g_ref[...], wg_ref[...], preferred_element_type=F32) + b_ref[...]
    q_ref[...] = jnp.dot(edges_ref[...], we_ref[...], preferred_element_type=F32) + ce


def _node_glob_kernel(nodes_ref, sp_ref, rp_ref, g_ref, wnn_ref, wns_ref,
                      wnr_ref, wng_ref, bn_ref, wgn_ref, wge_ref, wgg_ref,
                      bg_ref, nn_ref, gout_ref, nsum_acc, esum_acc):
    i = pl.program_id(0)
    s_agg = sp_ref[0] + sp_ref[1]
    r_agg = rp_ref[0] + rp_ref[1]
    cn = jnp.dot(g_ref[...], wng_ref[...], preferred_element_type=F32) + bn_ref[...]
    x = (jnp.dot(nodes_ref[...], wnn_ref[...], preferred_element_type=F32)
         + jnp.dot(s_agg, wns_ref[...], preferred_element_type=F32)
         + jnp.dot(r_agg, wnr_ref[...], preferred_element_type=F32)
         + cn)
    nn = jnp.maximum(x, 0.0)
    nn_ref[...] = nn

    @pl.when(i == 0)
    def _():
        nsum_acc[...] = jnp.zeros_like(nsum_acc)
        esum_acc[...] = jnp.zeros_like(esum_acc)

    nsum_acc[...] += jnp.sum(nn, axis=0, keepdims=True)
    esum_acc[...] += jnp.sum(s_agg, axis=0, keepdims=True)

    @pl.when(i == pl.num_programs(0) - 1)
    def _():
        gi = (jnp.dot(nsum_acc[...], wgn_ref[...], preferred_element_type=F32)
              + jnp.dot(esum_acc[...], wge_ref[...], preferred_element_type=F32)
              + jnp.dot(g_ref[...], wgg_ref[...], preferred_element_type=F32)
              + bg_ref[...])
        gout_ref[...] = jnp.maximum(gi, 0.0)


_C = 64
_NSUB = 16
_NCORE = 2


def _pad_nodes(n):
    step = 8 * _NSUB
    return ((n + step - 1) // step) * step


def _zero_vmem_rows(buf, nrows, ncols):
    def row(i, _):
        for j in range(ncols // 16):
            buf[i, pl.ds(j * 16, 16)] = jnp.zeros((16,), F32)
        return _
    lax.fori_loop(0, nrows, row, None)


def _tile_chunks(wid, total_chunks):
    nbase = total_chunks // (_NCORE * _NSUB)
    rem = total_chunks - nbase * _NCORE * _NSUB
    extra = jnp.where(wid < rem, 1, 0)
    start = wid * nbase + jnp.minimum(wid, rem)
    return start, nbase + extra


def _zero_agg_slice(buf, agg, sid):
    rows_per_tile = agg.shape[0] // _NSUB
    zb = sid * rows_per_tile
    _zero_vmem_rows(buf, _C, 128)
    nfull = rows_per_tile // _C
    tail = rows_per_tile - nfull * _C
    for t in range(nfull):
        pltpu.sync_copy(buf, agg.at[pl.ds(zb + t * _C, _C)])
    if tail:
        pltpu.sync_copy(buf.at[pl.ds(0, tail)], agg.at[pl.ds(zb + nfull * _C, tail)])


def _relu_sum3(qb, psb, prb):
    def row(i, _):
        for u in range(2):
            for j in range(8):
                sl = pl.ds(j * 16, 16)
                qb[2 * i + u, sl] = jnp.maximum(
                    qb[2 * i + u, sl] + psb[2 * i + u, sl] + prb[2 * i + u, sl],
                    0.0)
        return _
    lax.fori_loop(0, _C // 2, row, None)


def _edge_sc_body(ps_hbm, pr_hbm, q_hbm, s_hbm, r_hbm, en_hbm, sp_hbm,
                  gis0, gis1, gir0, gir1, sis0, sis1,
                  qb0, qb1, psb0, psb1, prb0, prb1, agg,
                  sem_in0, sem_in1, sem_out0, sem_out1,
                  sem_gi0, sem_gi1, sem_si0, sem_si1):
    cid = lax.axis_index("c")
    sid = lax.axis_index("s")
    wid = cid * _NSUB + sid
    rows_per_tile = agg.shape[0] // _NSUB
    zb = sid * rows_per_tile

    gis = (gis0, gis1)
    gir = (gir0, gir1)
    sis = (sis0, sis1)
    qb = (qb0, qb1)
    psb = (psb0, psb1)
    prb = (prb0, prb1)
    sem_in = (sem_in0, sem_in1)
    sem_out = (sem_out0, sem_out1)
    sem_gi = (sem_gi0, sem_gi1)
    sem_si = (sem_si0, sem_si1)

    total_chunks = q_hbm.shape[0] // _C
    start, nch = _tile_chunks(wid, total_chunks)
    ns = total_chunks // (_NCORE * _NSUB) + 1

    def issue_gidx(k, b):
        base = (start + k) * _C
        pltpu.async_copy(s_hbm.at[pl.ds(base, _C)], gis[b], sem_gi[b])
        pltpu.async_copy(r_hbm.at[pl.ds(base, _C)], gir[b], sem_gi[b])

    def issue_sidx(k, b):
        base = (start + k) * _C
        pltpu.async_copy(s_hbm.at[pl.ds(base, _C)], sis[b], sem_si[b])

    def issue_data(k, b):
        base = (start + k) * _C
        pltpu.async_copy(q_hbm.at[pl.ds(base, _C)], qb[b], sem_in[b])
        pltpu.async_copy(ps_hbm.at[gis[b]], psb[b], sem_in[b])
        pltpu.async_copy(pr_hbm.at[gir[b]], prb[b], sem_in[b])

    def drain_gidx(b):
        pltpu.make_async_copy(s_hbm.at[pl.ds(0, _C)], gis[b], sem_gi[b]).wait()
        pltpu.make_async_copy(r_hbm.at[pl.ds(0, _C)], gir[b], sem_gi[b]).wait()

    def drain_sidx(b):
        pltpu.make_async_copy(s_hbm.at[pl.ds(0, _C)], sis[b], sem_si[b]).wait()

    def drain_data(b):
        pltpu.make_async_copy(q_hbm.at[pl.ds(0, _C)], qb[b], sem_in[b]).wait()
        pltpu.make_async_copy(q_hbm.at[pl.ds(0, _C)], psb[b], sem_in[b]).wait()
        pltpu.make_async_copy(q_hbm.at[pl.ds(0, _C)], prb[b], sem_in[b]).wait()

    def drain_out(b):
        pltpu.make_async_copy(qb[b], en_hbm.at[pl.ds(0, _C)], sem_out[b]).wait()

    _zero_agg_slice(qb0, agg, sid)
    issue_gidx(0, 0)
    issue_gidx(1, 1)
    issue_sidx(0, 0)
    issue_sidx(1, 1)
    plsc.subcore_barrier()
    drain_gidx(0)
    issue_data(0, 0)

    def step(g, _):
        for j in range(2):
            b, ob = j, 1 - j
            k = 2 * g + j
            @pl.when(k < nch)
            def _():
                drain_data(b)
            @pl.when(k + 2 < nch)
            def _():
                issue_gidx(k + 2, b)
            @pl.when(k + 1 < nch)
            def _():
                drain_gidx(ob)
            @pl.when((k >= 1) & (k <= nch))
            def _():
                drain_out(ob)
            @pl.when(k + 1 < nch)
            def _():
                issue_data(k + 1, ob)
            @pl.when(k < nch)
            def _():
                _relu_sum3(qb[b], psb[b], prb[b])
                base = (start + k) * _C
                pltpu.async_copy(qb[b], en_hbm.at[pl.ds(base, _C)], sem_out[b])
                drain_sidx(b)
                pltpu.sync_copy(qb[b], agg.at[sis[b]], add=True)
            @pl.when(k + 2 < nch)
            def _():
                issue_sidx(k + 2, b)
        return _

    lax.fori_loop(0, (ns + 2) // 2, step, None)

    plsc.subcore_barrier()
    pltpu.sync_copy(agg.at[pl.ds(zb, rows_per_tile)],
                    sp_hbm.at[cid, pl.ds(zb, rows_per_tile)])


def _recv_sc_body(en_hbm, r_hbm, rp_hbm, ir0, ir1, buf0, buf1, agg,
                  sem_in0, sem_in1):
    cid = lax.axis_index("c")
    sid = lax.axis_index("s")
    wid = cid * _NSUB + sid
    rows_per_tile = agg.shape[0] // _NSUB
    zb = sid * rows_per_tile

    ir = (ir0, ir1)
    buf = (buf0, buf1)
    sem_in = (sem_in0, sem_in1)

    total_chunks = en_hbm.shape[0] // _C
    start, nch = _tile_chunks(wid, total_chunks)
    ns = total_chunks // (_NCORE * _NSUB) + 1

    def issue_data(k, b):
        base = (start + k) * _C
        pltpu.async_copy(r_hbm.at[pl.ds(base, _C)], ir[b], sem_in[b])
        pltpu.async_copy(en_hbm.at[pl.ds(base, _C)], buf[b], sem_in[b])

    def drain_data(b):
        pltpu.make_async_copy(r_hbm.at[pl.ds(0, _C)], ir[b], sem_in[b]).wait()
        pltpu.make_async_copy(en_hbm.at[pl.ds(0, _C)], buf[b], sem_in[b]).wait()

    _zero_agg_slice(buf0, agg, sid)
    issue_data(0, 0)
    plsc.subcore_barrier()

    def step(g, _):
        for j in range(2):
            b, ob = j, 1 - j
            k = 2 * g + j

            @pl.when(k + 1 < nch)
            def _():
                issue_data(k + 1, ob)

            @pl.when(k < nch)
            def _():
                drain_data(b)
                pltpu.sync_copy(buf[b], agg.at[ir[b]], add=True)
        return _

    lax.fori_loop(0, (ns + 2) // 2, step, None)
    plsc.subcore_barrier()
    pltpu.sync_copy(agg.at[pl.ds(zb, rows_per_tile)],
                    rp_hbm.at[cid, pl.ds(zb, rows_per_tile)])


def kernel(nodes, edges, senders, receivers, globals_, n_node, n_edge,
           W_edge, b_edge, W_node, b_node, W_glob, b_glob):
    N, F = nodes.shape
    E, DE = edges.shape
    DG = globals_.shape[1]
    DEO = b_edge.shape[0]
    DNO = b_node.shape[0]

    senders = senders.astype(jnp.int32)
    receivers = receivers.astype(jnp.int32)

    we_e = W_edge[:DE]
    we_s = W_edge[DE:DE + F]
    we_r = W_edge[DE + F:DE + 2 * F]
    we_g = W_edge[DE + 2 * F:]
    wn_n = W_node[:F]
    wn_s = W_node[F:F + DEO]
    wn_r = W_node[F + DEO:F + 2 * DEO]
    wn_g = W_node[F + 2 * DEO:]
    wg_n = W_glob[:DNO]
    wg_e = W_glob[DNO:DNO + DEO]
    wg_g = W_glob[DNO + DEO:]
    b_edge2 = b_edge.reshape(1, DEO)
    b_node2 = b_node.reshape(1, DNO)
    b_glob2 = b_glob.reshape(1, -1)

    ps, pr = pl.pallas_call(
        _proj_kernel,
        out_shape=(jax.ShapeDtypeStruct((N, DEO), F32),
                   jax.ShapeDtypeStruct((N, DEO), F32)),
    )(nodes, we_s, we_r)

    BE = 8000
    q = pl.pallas_call(
        _edge_pre_kernel,
        grid=(E // BE,),
        in_specs=[
            pl.BlockSpec((BE, DE), lambda i: (i, 0)),
            pl.BlockSpec((DE, DEO), lambda i: (0, 0)),
            pl.BlockSpec((1, DG), lambda i: (0, 0)),
            pl.BlockSpec((DG, DEO), lambda i: (0, 0)),
            pl.BlockSpec((1, DEO), lambda i: (0, 0)),
        ],
        out_specs=pl.BlockSpec((BE, DEO), lambda i: (i, 0)),
        out_shape=jax.ShapeDtypeStruct((E, DEO), F32),
    )(edges, we_e, globals_, we_g, b_edge2)

    NP = _pad_nodes(N)
    mesh = plsc.VectorSubcoreMesh(core_axis_name="c", subcore_axis_name="s")
    edges_new, sent_part = pl.kernel(
        _edge_sc_body,
        out_type=(jax.ShapeDtypeStruct((E, DEO), F32),
                  jax.ShapeDtypeStruct((_NCORE, NP, DEO), F32)),
        mesh=mesh,
        scratch_types=(
            [pltpu.VMEM((_C,), jnp.int32)] * 6
            + [pltpu.VMEM((_C, DEO), F32)] * 6
            + [pltpu.VMEM_SHARED((NP, DEO), F32)]
            + [pltpu.SemaphoreType.DMA] * 8
        ),
    )(ps, pr, q, senders, receivers)

    recv_part = pl.kernel(
        _recv_sc_body,
        out_type=jax.ShapeDtypeStruct((_NCORE, NP, DEO), F32),
        mesh=mesh,
        scratch_types=(
            [pltpu.VMEM((_C,), jnp.int32)] * 2
            + [pltpu.VMEM((_C, DEO), F32)] * 2
            + [pltpu.VMEM_SHARED((NP, DEO), F32)]
            + [pltpu.SemaphoreType.DMA] * 2
        ),
    )(edges_new, receivers)

    BN = 1000
    nodes_new, globals_new = pl.pallas_call(
        _node_glob_kernel,
        grid=(N // BN,),
        in_specs=[
            pl.BlockSpec((BN, F), lambda i: (i, 0)),
            pl.BlockSpec((_NCORE, BN, DEO), lambda i: (0, i, 0)),
            pl.BlockSpec((_NCORE, BN, DEO), lambda i: (0, i, 0)),
            pl.BlockSpec((1, DG), lambda i: (0, 0)),
            pl.BlockSpec((F, DNO), lambda i: (0, 0)),
            pl.BlockSpec((DEO, DNO), lambda i: (0, 0)),
            pl.BlockSpec((DEO, DNO), lambda i: (0, 0)),
            pl.BlockSpec((DG, DNO), lambda i: (0, 0)),
            pl.BlockSpec((1, DNO), lambda i: (0, 0)),
            pl.BlockSpec((DNO, b_glob.shape[0]), lambda i: (0, 0)),
            pl.BlockSpec((DEO, b_glob.shape[0]), lambda i: (0, 0)),
            pl.BlockSpec((DG, b_glob.shape[0]), lambda i: (0, 0)),
            pl.BlockSpec((1, b_glob.shape[0]), lambda i: (0, 0)),
        ],
        out_specs=(pl.BlockSpec((BN, DNO), lambda i: (i, 0)),
                   pl.BlockSpec((1, b_glob.shape[0]), lambda i: (0, 0))),
        out_shape=(jax.ShapeDtypeStruct((N, DNO), F32),
                   jax.ShapeDtypeStruct((1, b_glob.shape[0]), F32)),
        scratch_shapes=[pltpu.VMEM((1, DNO), F32),
                        pltpu.VMEM((1, DEO), F32)],
    )(nodes, sent_part, recv_part, globals_, wn_n, wn_s, wn_r, wn_g,
      b_node2, wg_n, wg_e, wg_g, b_glob2)

    return nodes_new, edges_new, globals_new


# speedup vs baseline: 6.6214x; 1.0857x over previous
import functools

import jax
import jax.numpy as jnp
import numpy as np
from jax import lax
from jax.experimental import pallas as pl
from jax.experimental.pallas import tpu as pltpu
from jax.experimental.pallas import tpu_sc as plsc

F32 = jnp.float32


def _edge_pre_kernel(edges_ref, we_ref, g_ref, wg_ref, b_ref, nodes_ref,
                     ws_ref, wr_ref, q_ref, ps_ref, pr_ref):
    ce = jnp.dot(g_ref[...], wg_ref[...], preferred_element_type=F32) + b_ref[...]
    q_ref[...] = jnp.dot(edges_ref[...], we_ref[...],
                         preferred_element_type=F32) + ce

    @pl.when(pl.program_id(0) == 0)
    def _():
        x = nodes_ref[...]
        ps_ref[...] = jnp.dot(x, ws_ref[...], preferred_element_type=F32)
        pr_ref[...] = jnp.dot(x, wr_ref[...], preferred_element_type=F32)


def _node_glob_kernel(nodes_ref, sp_ref, rp_ref, g_ref, wnn_ref, wns_ref,
                      wnr_ref, wng_ref, bn_ref, wgn_ref, wge_ref, wgg_ref,
                      bg_ref, nn_ref, gout_ref, nsum_acc, esum_acc):
    i = pl.program_id(0)
    s_agg = sp_ref[0] + sp_ref[1]
    r_agg = rp_ref[0] + rp_ref[1]
    cn = jnp.dot(g_ref[...], wng_ref[...], preferred_element_type=F32) + bn_ref[...]
    x = (jnp.dot(nodes_ref[...], wnn_ref[...], preferred_element_type=F32)
         + jnp.dot(s_agg, wns_ref[...], preferred_element_type=F32)
         + jnp.dot(r_agg, wnr_ref[...], preferred_element_type=F32)
         + cn)
    nn = jnp.maximum(x, 0.0)
    nn_ref[...] = nn

    @pl.when(i == 0)
    def _():
        nsum_acc[...] = jnp.zeros_like(nsum_acc)
        esum_acc[...] = jnp.zeros_like(esum_acc)

    nsum_acc[...] += jnp.sum(nn, axis=0, keepdims=True)
    esum_acc[...] += jnp.sum(s_agg, axis=0, keepdims=True)

    @pl.when(i == pl.num_programs(0) - 1)
    def _():
        gi = (jnp.dot(nsum_acc[...], wgn_ref[...], preferred_element_type=F32)
              + jnp.dot(esum_acc[...], wge_ref[...], preferred_element_type=F32)
              + jnp.dot(g_ref[...], wgg_ref[...], preferred_element_type=F32)
              + bg_ref[...])
        gout_ref[...] = jnp.maximum(gi, 0.0)


_C = 80
_NSUB = 16
_NCORE = 2


def _pad_nodes(n):
    step = 8 * _NSUB
    return ((n + step - 1) // step) * step


def _zero_vmem_rows(buf, nrows, ncols):
    def row(i, _):
        for j in range(ncols // 16):
            buf[i, pl.ds(j * 16, 16)] = jnp.zeros((16,), F32)
        return _
    lax.fori_loop(0, nrows, row, None)


def _tile_chunks(wid, total_chunks):
    nbase = total_chunks // (_NCORE * _NSUB)
    rem = total_chunks - nbase * _NCORE * _NSUB
    extra = jnp.where(wid < rem, 1, 0)
    start = wid * nbase + jnp.minimum(wid, rem)
    return start, nbase + extra


def _zero_agg_slice(buf, agg, sid):
    rows_per_tile = agg.shape[0] // _NSUB
    zb = sid * rows_per_tile
    _zero_vmem_rows(buf, _C, 128)
    nfull = rows_per_tile // _C
    tail = rows_per_tile - nfull * _C
    for t in range(nfull):
        pltpu.sync_copy(buf, agg.at[pl.ds(zb + t * _C, _C)])
    if tail:
        pltpu.sync_copy(buf.at[pl.ds(0, tail)], agg.at[pl.ds(zb + nfull * _C, tail)])


def _relu_inplace(qb):
    def row(i, _):
        for u in range(2):
            r = 2 * i + u
            for j in range(8):
                sl = pl.ds(j * 16, 16)
                qb[r, sl] = jnp.maximum(qb[r, sl], 0.0)
        return _
    lax.fori_loop(0, _C // 2, row, None)


def _edge_sc_body(ps_hbm, pr_hbm, q_hbm, s_hbm, r_hbm, en_hbm, sp_hbm,
                  gis0, gis1, gis2, gir0, gir1, gir2, sis0, sis1, sis2,
                  qb0, qb1, qb2, agg,
                  sem_q0, sem_q1, sem_q2, sem_in0, sem_in1, sem_in2,
                  sem_out0, sem_out1, sem_out2,
                  sem_gi0, sem_gi1, sem_gi2, sem_si0, sem_si1, sem_si2):
    cid = lax.axis_index("c")
    sid = lax.axis_index("s")
    wid = cid * _NSUB + sid
    rows_per_tile = agg.shape[0] // _NSUB
    zb = sid * rows_per_tile

    gis = (gis0, gis1, gis2)
    gir = (gir0, gir1, gir2)
    sis = (sis0, sis1, sis2)
    qb = (qb0, qb1, qb2)
    sem_q = (sem_q0, sem_q1, sem_q2)
    sem_in = (sem_in0, sem_in1, sem_in2)
    sem_out = (sem_out0, sem_out1, sem_out2)
    sem_gi = (sem_gi0, sem_gi1, sem_gi2)
    sem_si = (sem_si0, sem_si1, sem_si2)

    total_chunks = q_hbm.shape[0] // _C
    start, nch = _tile_chunks(wid, total_chunks)
    ns = total_chunks // (_NCORE * _NSUB) + 1

    def issue_gidx(k, b):
        base = (start + k) * _C
        pltpu.async_copy(s_hbm.at[pl.ds(base, _C)], gis[b], sem_gi[b])
        pltpu.async_copy(r_hbm.at[pl.ds(base, _C)], gir[b], sem_gi[b])

    def issue_sidx(k, b):
        base = (start + k) * _C
        pltpu.async_copy(s_hbm.at[pl.ds(base, _C)], sis[b], sem_si[b])

    def issue_q(k, b):
        base = (start + k) * _C
        pltpu.async_copy(q_hbm.at[pl.ds(base, _C)], qb[b], sem_q[b])

    def issue_gadds(b):
        pltpu.async_copy(ps_hbm.at[gis[b]], qb[b], sem_in[b], add=True)
        pltpu.async_copy(pr_hbm.at[gir[b]], qb[b], sem_in[b], add=True)

    def drain_gidx(b):
        pltpu.make_async_copy(s_hbm.at[pl.ds(0, _C)], gis[b], sem_gi[b]).wait()
        pltpu.make_async_copy(r_hbm.at[pl.ds(0, _C)], gir[b], sem_gi[b]).wait()

    def drain_sidx(b):
        pltpu.make_async_copy(s_hbm.at[pl.ds(0, _C)], sis[b], sem_si[b]).wait()

    def drain_q(b):
        pltpu.make_async_copy(q_hbm.at[pl.ds(0, _C)], qb[b], sem_q[b]).wait()

    def drain_gadds(b):
        pltpu.make_async_copy(q_hbm.at[pl.ds(0, _C)], qb[b], sem_in[b]).wait()
        pltpu.make_async_copy(q_hbm.at[pl.ds(0, _C)], qb[b], sem_in[b]).wait()

    def drain_out(b):
        pltpu.make_async_copy(qb[b], en_hbm.at[pl.ds(0, _C)], sem_out[b]).wait()

    _zero_agg_slice(qb0, agg, sid)
    issue_gidx(0, 0)
    issue_gidx(1, 1)
    issue_q(0, 0)
    issue_q(1, 1)
    issue_sidx(0, 0)
    issue_sidx(1, 1)
    issue_sidx(2, 2)
    plsc.subcore_barrier()
    drain_gidx(0)
    drain_q(0)
    issue_gadds(0)

    def step(g, _):
        for j in range(3):
            b = j
            b1 = (j + 1) % 3
            b2 = (j + 2) % 3
            k = 3 * g + j
            @pl.when(k + 1 < nch)
            def _():
                drain_gidx(b1)
                drain_q(b1)
                issue_gadds(b1)
            @pl.when((k >= 1) & (k <= nch))
            def _():
                drain_out(b2)
            @pl.when(k + 2 < nch)
            def _():
                issue_gidx(k + 2, b2)
                issue_q(k + 2, b2)
            @pl.when(k < nch)
            def _():
                drain_gadds(b)
                _relu_inplace(qb[b])
                base = (start + k) * _C
                pltpu.async_copy(qb[b], en_hbm.at[pl.ds(base, _C)], sem_out[b])
                drain_sidx(b)
                pltpu.sync_copy(qb[b], agg.at[sis[b]], add=True)
            @pl.when(k + 3 < nch)
            def _():
                issue_sidx(k + 3, b)
        return _

    lax.fori_loop(0, (ns + 3) // 3, step, None)

    plsc.subcore_barrier()
    pltpu.sync_copy(agg.at[pl.ds(zb, rows_per_tile)],
                    sp_hbm.at[cid, pl.ds(zb, rows_per_tile)])


def _recv_sc_body(en_hbm, r_hbm, rp_hbm, ir0, ir1, buf0, buf1, agg,
                  sem_in0, sem_in1):
    cid = lax.axis_index("c")
    sid = lax.axis_index("s")
    wid = cid * _NSUB + sid
    rows_per_tile = agg.shape[0] // _NSUB
    zb = sid * rows_per_tile

    ir = (ir0, ir1)
    buf = (buf0, buf1)
    sem_in = (sem_in0, sem_in1)

    total_chunks = en_hbm.shape[0] // _C
    start, nch = _tile_chunks(wid, total_chunks)
    ns = total_chunks // (_NCORE * _NSUB) + 1

    def issue_data(k, b):
        base = (start + k) * _C
        pltpu.async_copy(r_hbm.at[pl.ds(base, _C)], ir[b], sem_in[b])
        pltpu.async_copy(en_hbm.at[pl.ds(base, _C)], buf[b], sem_in[b])

    def drain_data(b):
        pltpu.make_async_copy(r_hbm.at[pl.ds(0, _C)], ir[b], sem_in[b]).wait()
        pltpu.make_async_copy(en_hbm.at[pl.ds(0, _C)], buf[b], sem_in[b]).wait()

    _zero_agg_slice(buf0, agg, sid)
    issue_data(0, 0)
    plsc.subcore_barrier()

    def step(g, _):
        for j in range(2):
            b, ob = j, 1 - j
            k = 2 * g + j

            @pl.when(k + 1 < nch)
            def _():
                issue_data(k + 1, ob)

            @pl.when(k < nch)
            def _():
                drain_data(b)
                pltpu.sync_copy(buf[b], agg.at[ir[b]], add=True)
        return _

    lax.fori_loop(0, (ns + 2) // 2, step, None)
    plsc.subcore_barrier()
    pltpu.sync_copy(agg.at[pl.ds(zb, rows_per_tile)],
                    rp_hbm.at[cid, pl.ds(zb, rows_per_tile)])


def kernel(nodes, edges, senders, receivers, globals_, n_node, n_edge,
           W_edge, b_edge, W_node, b_node, W_glob, b_glob):
    N, F = nodes.shape
    E, DE = edges.shape
    DG = globals_.shape[1]
    DEO = b_edge.shape[0]
    DNO = b_node.shape[0]

    senders = senders.astype(jnp.int32)
    receivers = receivers.astype(jnp.int32)

    we_e = W_edge[:DE]
    we_s = W_edge[DE:DE + F]
    we_r = W_edge[DE + F:DE + 2 * F]
    we_g = W_edge[DE + 2 * F:]
    wn_n = W_node[:F]
    wn_s = W_node[F:F + DEO]
    wn_r = W_node[F + DEO:F + 2 * DEO]
    wn_g = W_node[F + 2 * DEO:]
    wg_n = W_glob[:DNO]
    wg_e = W_glob[DNO:DNO + DEO]
    wg_g = W_glob[DNO + DEO:]
    b_edge2 = b_edge.reshape(1, DEO)
    b_node2 = b_node.reshape(1, DNO)
    b_glob2 = b_glob.reshape(1, -1)

    BE = 8000
    q, ps, pr = pl.pallas_call(
        _edge_pre_kernel,
        grid=(E // BE,),
        in_specs=[
            pl.BlockSpec((BE, DE), lambda i: (i, 0)),
            pl.BlockSpec((DE, DEO), lambda i: (0, 0)),
            pl.BlockSpec((1, DG), lambda i: (0, 0)),
            pl.BlockSpec((DG, DEO), lambda i: (0, 0)),
            pl.BlockSpec((1, DEO), lambda i: (0, 0)),
            pl.BlockSpec((N, F), lambda i: (0, 0)),
            pl.BlockSpec((F, DEO), lambda i: (0, 0)),
            pl.BlockSpec((F, DEO), lambda i: (0, 0)),
        ],
        out_specs=(pl.BlockSpec((BE, DEO), lambda i: (i, 0)),
                   pl.BlockSpec((N, DEO), lambda i: (0, 0)),
                   pl.BlockSpec((N, DEO), lambda i: (0, 0))),
        out_shape=(jax.ShapeDtypeStruct((E, DEO), F32),
                   jax.ShapeDtypeStruct((N, DEO), F32),
                   jax.ShapeDtypeStruct((N, DEO), F32)),
    )(edges, we_e, globals_, we_g, b_edge2, nodes, we_s, we_r)

    NP = _pad_nodes(N)
    mesh = plsc.VectorSubcoreMesh(core_axis_name="c", subcore_axis_name="s")
    edges_new, sent_part = pl.kernel(
        _edge_sc_body,
        out_type=(jax.ShapeDtypeStruct((E, DEO), F32),
                  jax.ShapeDtypeStruct((_NCORE, NP, DEO), F32)),
        mesh=mesh,
        scratch_types=(
            [pltpu.VMEM((_C,), jnp.int32)] * 9
            + [pltpu.VMEM((_C, DEO), F32)] * 3
            + [pltpu.VMEM_SHARED((NP, DEO), F32)]
            + [pltpu.SemaphoreType.DMA] * 15
        ),
    )(ps, pr, q, senders, receivers)

    recv_part = pl.kernel(
        _recv_sc_body,
        out_type=jax.ShapeDtypeStruct((_NCORE, NP, DEO), F32),
        mesh=mesh,
        scratch_types=(
            [pltpu.VMEM((_C,), jnp.int32)] * 2
            + [pltpu.VMEM((_C, DEO), F32)] * 2
            + [pltpu.VMEM_SHARED((NP, DEO), F32)]
            + [pltpu.SemaphoreType.DMA] * 2
        ),
    )(edges_new, receivers)

    BN = 1000
    nodes_new, globals_new = pl.pallas_call(
        _node_glob_kernel,
        grid=(N // BN,),
        in_specs=[
            pl.BlockSpec((BN, F), lambda i: (i, 0)),
            pl.BlockSpec((_NCORE, BN, DEO), lambda i: (0, i, 0)),
            pl.BlockSpec((_NCORE, BN, DEO), lambda i: (0, i, 0)),
            pl.BlockSpec((1, DG), lambda i: (0, 0)),
            pl.BlockSpec((F, DNO), lambda i: (0, 0)),
            pl.BlockSpec((DEO, DNO), lambda i: (0, 0)),
            pl.BlockSpec((DEO, DNO), lambda i: (0, 0)),
            pl.BlockSpec((DG, DNO), lambda i: (0, 0)),
            pl.BlockSpec((1, DNO), lambda i: (0, 0)),
            pl.BlockSpec((DNO, b_glob.shape[0]), lambda i: (0, 0)),
            pl.BlockSpec((DEO, b_glob.shape[0]), lambda i: (0, 0)),
            pl.BlockSpec((DG, b_glob.shape[0]), lambda i: (0, 0)),
            pl.BlockSpec((1, b_glob.shape[0]), lambda i: (0, 0)),
        ],
        out_specs=(pl.BlockSpec((BN, DNO), lambda i: (i, 0)),
                   pl.BlockSpec((1, b_glob.shape[0]), lambda i: (0, 0))),
        out_shape=(jax.ShapeDtypeStruct((N, DNO), F32),
                   jax.ShapeDtypeStruct((1, b_glob.shape[0]), F32)),
        scratch_shapes=[pltpu.VMEM((1, DNO), F32),
                        pltpu.VMEM((1, DEO), F32)],
    )(nodes, sent_part, recv_part, globals_, wn_n, wn_s, wn_r, wn_g,
      b_node2, wg_n, wg_e, wg_g, b_glob2)

    return nodes_new, edges_new, globals_new


# speedup vs baseline: 6.8308x; 1.0316x over previous
import functools

import jax
import jax.numpy as jnp
import numpy as np
from jax import lax
from jax.experimental import pallas as pl
from jax.experimental.pallas import tpu as pltpu
from jax.experimental.pallas import tpu_sc as plsc

F32 = jnp.float32


def _edge_pre_kernel(edges_ref, we_ref, g_ref, wg_ref, b_ref, nodes_ref,
                     ws_ref, wr_ref, q_ref, ps_ref, pr_ref):
    ce = jnp.dot(g_ref[...], wg_ref[...], preferred_element_type=F32) + b_ref[...]
    q_ref[...] = jnp.dot(edges_ref[...], we_ref[...],
                         preferred_element_type=F32) + ce

    @pl.when(pl.program_id(0) == 0)
    def _():
        x = nodes_ref[...]
        ps_ref[...] = jnp.dot(x, ws_ref[...], preferred_element_type=F32)
        pr_ref[...] = jnp.dot(x, wr_ref[...], preferred_element_type=F32)


def _node_glob_kernel(nodes_ref, sp_ref, rp_ref, g_ref, wnn_ref, wns_ref,
                      wnr_ref, wng_ref, bn_ref, wgn_ref, wge_ref, wgg_ref,
                      bg_ref, nn_ref, gout_ref, nsum_acc, esum_acc):
    i = pl.program_id(0)
    s_agg = sp_ref[0] + sp_ref[1]
    r_agg = rp_ref[0] + rp_ref[1]
    cn = jnp.dot(g_ref[...], wng_ref[...], preferred_element_type=F32) + bn_ref[...]
    x = (jnp.dot(nodes_ref[...], wnn_ref[...], preferred_element_type=F32)
         + jnp.dot(s_agg, wns_ref[...], preferred_element_type=F32)
         + jnp.dot(r_agg, wnr_ref[...], preferred_element_type=F32)
         + cn)
    nn = jnp.maximum(x, 0.0)
    nn_ref[...] = nn

    @pl.when(i == 0)
    def _():
        nsum_acc[...] = jnp.zeros_like(nsum_acc)
        esum_acc[...] = jnp.zeros_like(esum_acc)

    nsum_acc[...] += jnp.sum(nn, axis=0, keepdims=True)
    esum_acc[...] += jnp.sum(s_agg, axis=0, keepdims=True)

    @pl.when(i == pl.num_programs(0) - 1)
    def _():
        gi = (jnp.dot(nsum_acc[...], wgn_ref[...], preferred_element_type=F32)
              + jnp.dot(esum_acc[...], wge_ref[...], preferred_element_type=F32)
              + jnp.dot(g_ref[...], wgg_ref[...], preferred_element_type=F32)
              + bg_ref[...])
        gout_ref[...] = jnp.maximum(gi, 0.0)


_C = 128
_NSUB = 16
_NCORE = 2


def _pad_nodes(n):
    step = 8 * _NSUB
    return ((n + step - 1) // step) * step


def _zero_vmem_rows(buf, nrows, ncols):
    def row(i, _):
        for j in range(ncols // 16):
            buf[i, pl.ds(j * 16, 16)] = jnp.zeros((16,), F32)
        return _
    lax.fori_loop(0, nrows, row, None)


def _tile_chunks(wid, total_chunks):
    nbase = total_chunks // (_NCORE * _NSUB)
    rem = total_chunks - nbase * _NCORE * _NSUB
    extra = jnp.where(wid < rem, 1, 0)
    start = wid * nbase + jnp.minimum(wid, rem)
    return start, nbase + extra


def _zero_agg_slice(buf, agg, sid):
    rows_per_tile = agg.shape[0] // _NSUB
    zb = sid * rows_per_tile
    _zero_vmem_rows(buf, _C, 128)
    nfull = rows_per_tile // _C
    tail = rows_per_tile - nfull * _C
    for t in range(nfull):
        pltpu.sync_copy(buf, agg.at[pl.ds(zb + t * _C, _C)])
    if tail:
        pltpu.sync_copy(buf.at[pl.ds(0, tail)], agg.at[pl.ds(zb + nfull * _C, tail)])


def _relu_inplace(qb):
    def row(i, _):
        for u in range(2):
            r = 2 * i + u
            for j in range(8):
                sl = pl.ds(j * 16, 16)
                qb[r, sl] = jnp.maximum(qb[r, sl], 0.0)
        return _
    lax.fori_loop(0, _C // 2, row, None)


def _edge_sc_body(ps_hbm, pr_hbm, q_hbm, s_hbm, r_hbm, en_hbm, sp_hbm,
                  gis0, gis1, gis2, gir0, gir1, gir2,
                  qb0, qb1, qb2, agg,
                  sem_q0, sem_q1, sem_q2, sem_in0, sem_in1, sem_in2,
                  sem_out0, sem_out1, sem_out2,
                  sem_gi0, sem_gi1, sem_gi2):
    cid = lax.axis_index("c")
    sid = lax.axis_index("s")
    wid = cid * _NSUB + sid
    rows_per_tile = agg.shape[0] // _NSUB
    zb = sid * rows_per_tile

    gis = (gis0, gis1, gis2)
    gir = (gir0, gir1, gir2)
    qb = (qb0, qb1, qb2)
    sem_q = (sem_q0, sem_q1, sem_q2)
    sem_in = (sem_in0, sem_in1, sem_in2)
    sem_out = (sem_out0, sem_out1, sem_out2)
    sem_gi = (sem_gi0, sem_gi1, sem_gi2)

    total_chunks = q_hbm.shape[0] // _C
    start, nch = _tile_chunks(wid, total_chunks)
    ns = total_chunks // (_NCORE * _NSUB) + 1

    def issue_gidx(k, b):
        base = (start + k) * _C
        pltpu.async_copy(s_hbm.at[pl.ds(base, _C)], gis[b], sem_gi[b])
        pltpu.async_copy(r_hbm.at[pl.ds(base, _C)], gir[b], sem_gi[b])

    def issue_q(k, b):
        base = (start + k) * _C
        pltpu.async_copy(q_hbm.at[pl.ds(base, _C)], qb[b], sem_q[b])

    def issue_gadds(b):
        pltpu.async_copy(ps_hbm.at[gis[b]], qb[b], sem_in[b], add=True)
        pltpu.async_copy(pr_hbm.at[gir[b]], qb[b], sem_in[b], add=True)

    def drain_gidx(b):
        pltpu.make_async_copy(s_hbm.at[pl.ds(0, _C)], gis[b], sem_gi[b]).wait()
        pltpu.make_async_copy(r_hbm.at[pl.ds(0, _C)], gir[b], sem_gi[b]).wait()

    def drain_q(b):
        pltpu.make_async_copy(q_hbm.at[pl.ds(0, _C)], qb[b], sem_q[b]).wait()

    def drain_gadds(b):
        pltpu.make_async_copy(q_hbm.at[pl.ds(0, _C)], qb[b], sem_in[b]).wait()
        pltpu.make_async_copy(q_hbm.at[pl.ds(0, _C)], qb[b], sem_in[b]).wait()

    def drain_out(b):
        pltpu.make_async_copy(qb[b], en_hbm.at[pl.ds(0, _C)], sem_out[b]).wait()

    _zero_agg_slice(qb0, agg, sid)
    issue_gidx(0, 0)
    issue_gidx(1, 1)
    issue_q(0, 0)
    issue_q(1, 1)
    plsc.subcore_barrier()
    drain_gidx(0)
    drain_q(0)
    issue_gadds(0)

    def step(g, _):
        for j in range(3):
            b = j
            b1 = (j + 1) % 3
            b2 = (j + 2) % 3
            k = 3 * g + j
            @pl.when(k + 1 < nch)
            def _():
                drain_gidx(b1)
                drain_q(b1)
                issue_gadds(b1)
            @pl.when((k >= 1) & (k <= nch))
            def _():
                drain_out(b2)
            @pl.when(k + 2 < nch)
            def _():
                issue_gidx(k + 2, b2)
                issue_q(k + 2, b2)
            @pl.when(k < nch)
            def _():
                drain_gadds(b)
                _relu_inplace(qb[b])
                base = (start + k) * _C
                pltpu.async_copy(qb[b], en_hbm.at[pl.ds(base, _C)], sem_out[b])
                pltpu.sync_copy(qb[b], agg.at[gis[b]], add=True)
        return _

    lax.fori_loop(0, (ns + 3) // 3, step, None)

    plsc.subcore_barrier()
    pltpu.sync_copy(agg.at[pl.ds(zb, rows_per_tile)],
                    sp_hbm.at[cid, pl.ds(zb, rows_per_tile)])


def _recv_sc_body(en_hbm, r_hbm, rp_hbm, ir0, ir1, buf0, buf1, agg,
                  sem_in0, sem_in1):
    cid = lax.axis_index("c")
    sid = lax.axis_index("s")
    wid = cid * _NSUB + sid
    rows_per_tile = agg.shape[0] // _NSUB
    zb = sid * rows_per_tile

    ir = (ir0, ir1)
    buf = (buf0, buf1)
    sem_in = (sem_in0, sem_in1)

    total_chunks = en_hbm.shape[0] // _C
    start, nch = _tile_chunks(wid, total_chunks)
    ns = total_chunks // (_NCORE * _NSUB) + 1

    def issue_data(k, b):
        base = (start + k) * _C
        pltpu.async_copy(r_hbm.at[pl.ds(base, _C)], ir[b], sem_in[b])
        pltpu.async_copy(en_hbm.at[pl.ds(base, _C)], buf[b], sem_in[b])

    def drain_data(b):
        pltpu.make_async_copy(r_hbm.at[pl.ds(0, _C)], ir[b], sem_in[b]).wait()
        pltpu.make_async_copy(en_hbm.at[pl.ds(0, _C)], buf[b], sem_in[b]).wait()

    _zero_agg_slice(buf0, agg, sid)
    issue_data(0, 0)
    plsc.subcore_barrier()

    def step(g, _):
        for j in range(2):
            b, ob = j, 1 - j
            k = 2 * g + j

            @pl.when(k + 1 < nch)
            def _():
                issue_data(k + 1, ob)

            @pl.when(k < nch)
            def _():
                drain_data(b)
                pltpu.sync_copy(buf[b], agg.at[ir[b]], add=True)
        return _

    lax.fori_loop(0, (ns + 2) // 2, step, None)
    plsc.subcore_barrier()
    pltpu.sync_copy(agg.at[pl.ds(zb, rows_per_tile)],
                    rp_hbm.at[cid, pl.ds(zb, rows_per_tile)])


def kernel(nodes, edges, senders, receivers, globals_, n_node, n_edge,
           W_edge, b_edge, W_node, b_node, W_glob, b_glob):
    N, F = nodes.shape
    E, DE = edges.shape
    DG = globals_.shape[1]
    DEO = b_edge.shape[0]
    DNO = b_node.shape[0]

    senders = senders.astype(jnp.int32)
    receivers = receivers.astype(jnp.int32)

    we_e = W_edge[:DE]
    we_s = W_edge[DE:DE + F]
    we_r = W_edge[DE + F:DE + 2 * F]
    we_g = W_edge[DE + 2 * F:]
    wn_n = W_node[:F]
    wn_s = W_node[F:F + DEO]
    wn_r = W_node[F + DEO:F + 2 * DEO]
    wn_g = W_node[F + 2 * DEO:]
    wg_n = W_glob[:DNO]
    wg_e = W_glob[DNO:DNO + DEO]
    wg_g = W_glob[DNO + DEO:]
    b_edge2 = b_edge.reshape(1, DEO)
    b_node2 = b_node.reshape(1, DNO)
    b_glob2 = b_glob.reshape(1, -1)

    BE = 8000
    q, ps, pr = pl.pallas_call(
        _edge_pre_kernel,
        grid=(E // BE,),
        in_specs=[
            pl.BlockSpec((BE, DE), lambda i: (i, 0)),
            pl.BlockSpec((DE, DEO), lambda i: (0, 0)),
            pl.BlockSpec((1, DG), lambda i: (0, 0)),
            pl.BlockSpec((DG, DEO), lambda i: (0, 0)),
            pl.BlockSpec((1, DEO), lambda i: (0, 0)),
            pl.BlockSpec((N, F), lambda i: (0, 0)),
            pl.BlockSpec((F, DEO), lambda i: (0, 0)),
            pl.BlockSpec((F, DEO), lambda i: (0, 0)),
        ],
        out_specs=(pl.BlockSpec((BE, DEO), lambda i: (i, 0)),
                   pl.BlockSpec((N, DEO), lambda i: (0, 0)),
                   pl.BlockSpec((N, DEO), lambda i: (0, 0))),
        out_shape=(jax.ShapeDtypeStruct((E, DEO), F32),
                   jax.ShapeDtypeStruct((N, DEO), F32),
                   jax.ShapeDtypeStruct((N, DEO), F32)),
    )(edges, we_e, globals_, we_g, b_edge2, nodes, we_s, we_r)

    NP = _pad_nodes(N)
    mesh = plsc.VectorSubcoreMesh(core_axis_name="c", subcore_axis_name="s")
    edges_new, sent_part = pl.kernel(
        _edge_sc_body,
        out_type=(jax.ShapeDtypeStruct((E, DEO), F32),
                  jax.ShapeDtypeStruct((_NCORE, NP, DEO), F32)),
        mesh=mesh,
        scratch_types=(
            [pltpu.VMEM((_C,), jnp.int32)] * 6
            + [pltpu.VMEM((_C, DEO), F32)] * 3
            + [pltpu.VMEM_SHARED((NP, DEO), F32)]
            + [pltpu.SemaphoreType.DMA] * 12
        ),
    )(ps, pr, q, senders, receivers)

    recv_part = pl.kernel(
        _recv_sc_body,
        out_type=jax.ShapeDtypeStruct((_NCORE, NP, DEO), F32),
        mesh=mesh,
        scratch_types=(
            [pltpu.VMEM((_C,), jnp.int32)] * 2
            + [pltpu.VMEM((_C, DEO), F32)] * 2
            + [pltpu.VMEM_SHARED((NP, DEO), F32)]
            + [pltpu.SemaphoreType.DMA] * 2
        ),
    )(edges_new, receivers)

    BN = 1000
    nodes_new, globals_new = pl.pallas_call(
        _node_glob_kernel,
        grid=(N // BN,),
        in_specs=[
            pl.BlockSpec((BN, F), lambda i: (i, 0)),
            pl.BlockSpec((_NCORE, BN, DEO), lambda i: (0, i, 0)),
            pl.BlockSpec((_NCORE, BN, DEO), lambda i: (0, i, 0)),
            pl.BlockSpec((1, DG), lambda i: (0, 0)),
            pl.BlockSpec((F, DNO), lambda i: (0, 0)),
            pl.BlockSpec((DEO, DNO), lambda i: (0, 0)),
            pl.BlockSpec((DEO, DNO), lambda i: (0, 0)),
            pl.BlockSpec((DG, DNO), lambda i: (0, 0)),
            pl.BlockSpec((1, DNO), lambda i: (0, 0)),
            pl.BlockSpec((DNO, b_glob.shape[0]), lambda i: (0, 0)),
            pl.BlockSpec((DEO, b_glob.shape[0]), lambda i: (0, 0)),
            pl.BlockSpec((DG, b_glob.shape[0]), lambda i: (0, 0)),
            pl.BlockSpec((1, b_glob.shape[0]), lambda i: (0, 0)),
        ],
        out_specs=(pl.BlockSpec((BN, DNO), lambda i: (i, 0)),
                   pl.BlockSpec((1, b_glob.shape[0]), lambda i: (0, 0))),
        out_shape=(jax.ShapeDtypeStruct((N, DNO), F32),
                   jax.ShapeDtypeStruct((1, b_glob.shape[0]), F32)),
        scratch_shapes=[pltpu.VMEM((1, DNO), F32),
                        pltpu.VMEM((1, DEO), F32)],
    )(nodes, sent_part, recv_part, globals_, wn_n, wn_s, wn_r, wn_g,
      b_node2, wg_n, wg_e, wg_g, b_glob2)

    return nodes_new, edges_new, globals_new


# speedup vs baseline: 8.3851x; 1.2275x over previous
import functools

import jax
import jax.numpy as jnp
import numpy as np
from jax import lax
from jax.experimental import pallas as pl
from jax.experimental.pallas import tpu as pltpu
from jax.experimental.pallas import tpu_sc as plsc

F32 = jnp.float32


def _edge_pre_kernel(edges_t_ref, we_ref, g_ref, wg_ref, b_ref, nodes_ref,
                     ws_ref, wr_ref, q_ref, ps_ref, pr_ref):
    ce = jnp.dot(g_ref[...], wg_ref[...], preferred_element_type=F32) + b_ref[...]
    q_ref[...] = lax.dot_general(
        edges_t_ref[...], we_ref[...],
        dimension_numbers=(((0,), (0,)), ((), ())),
        preferred_element_type=F32) + ce

    @pl.when(pl.program_id(0) == 0)
    def _():
        x = nodes_ref[...]
        ps_ref[...] = jnp.dot(x, ws_ref[...], preferred_element_type=F32)
        pr_ref[...] = jnp.dot(x, wr_ref[...], preferred_element_type=F32)


def _node_glob_kernel(nodes_ref, sp_ref, rp_ref, g_ref, wnn_ref, wns_ref,
                      wnr_ref, wng_ref, bn_ref, wgn_ref, wge_ref, wgg_ref,
                      bg_ref, nn_ref, gout_ref, nsum_acc, esum_acc):
    i = pl.program_id(0)
    s_agg = sp_ref[0] + sp_ref[1]
    r_agg = rp_ref[0] + rp_ref[1]
    cn = jnp.dot(g_ref[...], wng_ref[...], preferred_element_type=F32) + bn_ref[...]
    x = (jnp.dot(nodes_ref[...], wnn_ref[...], preferred_element_type=F32)
         + jnp.dot(s_agg, wns_ref[...], preferred_element_type=F32)
         + jnp.dot(r_agg, wnr_ref[...], preferred_element_type=F32)
         + cn)
    nn = jnp.maximum(x, 0.0)
    nn_ref[...] = nn

    @pl.when(i == 0)
    def _():
        nsum_acc[...] = jnp.zeros_like(nsum_acc)
        esum_acc[...] = jnp.zeros_like(esum_acc)

    nsum_acc[...] += jnp.sum(nn, axis=0, keepdims=True)
    esum_acc[...] += jnp.sum(s_agg, axis=0, keepdims=True)

    @pl.when(i == pl.num_programs(0) - 1)
    def _():
        gi = (jnp.dot(nsum_acc[...], wgn_ref[...], preferred_element_type=F32)
              + jnp.dot(esum_acc[...], wge_ref[...], preferred_element_type=F32)
              + jnp.dot(g_ref[...], wgg_ref[...], preferred_element_type=F32)
              + bg_ref[...])
        gout_ref[...] = jnp.maximum(gi, 0.0)


_C = 128
_NSUB = 16
_NCORE = 2


def _pad_nodes(n):
    step = 8 * _NSUB
    return ((n + step - 1) // step) * step


def _zero_vmem_rows(buf, nrows, ncols):
    def row(i, _):
        for j in range(ncols // 16):
            buf[i, pl.ds(j * 16, 16)] = jnp.zeros((16,), F32)
        return _
    lax.fori_loop(0, nrows, row, None)


def _tile_chunks(wid, total_chunks):
    nbase = total_chunks // (_NCORE * _NSUB)
    rem = total_chunks - nbase * _NCORE * _NSUB
    extra = jnp.where(wid < rem, 1, 0)
    start = wid * nbase + jnp.minimum(wid, rem)
    return start, nbase + extra


def _zero_agg_slice(buf, agg, sid):
    rows_per_tile = agg.shape[0] // _NSUB
    zb = sid * rows_per_tile
    _zero_vmem_rows(buf, _C, 128)
    nfull = rows_per_tile // _C
    tail = rows_per_tile - nfull * _C
    for t in range(nfull):
        pltpu.sync_copy(buf, agg.at[pl.ds(zb + t * _C, _C)])
    if tail:
        pltpu.sync_copy(buf.at[pl.ds(0, tail)], agg.at[pl.ds(zb + nfull * _C, tail)])


def _relu_inplace(qb):
    def row(i, _):
        for u in range(2):
            r = 2 * i + u
            for j in range(8):
                sl = pl.ds(j * 16, 16)
                qb[r, sl] = jnp.maximum(qb[r, sl], 0.0)
        return _
    lax.fori_loop(0, _C // 2, row, None)


def _edge_sc_body(ps_hbm, pr_hbm, q_hbm, s_hbm, r_hbm, en_hbm, sp_hbm,
                  gis0, gis1, gis2, gir0, gir1, gir2,
                  qb0, qb1, qb2, agg,
                  sem_q0, sem_q1, sem_q2, sem_in0, sem_in1, sem_in2,
                  sem_out0, sem_out1, sem_out2,
                  sem_gi0, sem_gi1, sem_gi2):
    cid = lax.axis_index("c")
    sid = lax.axis_index("s")
    wid = cid * _NSUB + sid
    rows_per_tile = agg.shape[0] // _NSUB
    zb = sid * rows_per_tile

    gis = (gis0, gis1, gis2)
    gir = (gir0, gir1, gir2)
    qb = (qb0, qb1, qb2)
    sem_q = (sem_q0, sem_q1, sem_q2)
    sem_in = (sem_in0, sem_in1, sem_in2)
    sem_out = (sem_out0, sem_out1, sem_out2)
    sem_gi = (sem_gi0, sem_gi1, sem_gi2)

    total_chunks = q_hbm.shape[0] // _C
    start, nch = _tile_chunks(wid, total_chunks)
    ns = total_chunks // (_NCORE * _NSUB) + 1

    def issue_gidx(k, b):
        base = (start + k) * _C
        pltpu.async_copy(s_hbm.at[pl.ds(base, _C)], gis[b], sem_gi[b])
        pltpu.async_copy(r_hbm.at[pl.ds(base, _C)], gir[b], sem_gi[b])

    def issue_q(k, b):
        base = (start + k) * _C
        pltpu.async_copy(q_hbm.at[pl.ds(base, _C)], qb[b], sem_q[b])

    def issue_gadds(b):
        pltpu.async_copy(ps_hbm.at[gis[b]], qb[b], sem_in[b], add=True)
        pltpu.async_copy(pr_hbm.at[gir[b]], qb[b], sem_in[b], add=True)

    def drain_gidx(b):
        pltpu.make_async_copy(s_hbm.at[pl.ds(0, _C)], gis[b], sem_gi[b]).wait()
        pltpu.make_async_copy(r_hbm.at[pl.ds(0, _C)], gir[b], sem_gi[b]).wait()

    def drain_q(b):
        pltpu.make_async_copy(q_hbm.at[pl.ds(0, _C)], qb[b], sem_q[b]).wait()

    def drain_gadds(b):
        pltpu.make_async_copy(q_hbm.at[pl.ds(0, _C)], qb[b], sem_in[b]).wait()
        pltpu.make_async_copy(q_hbm.at[pl.ds(0, _C)], qb[b], sem_in[b]).wait()

    def drain_out(b):
        pltpu.make_async_copy(qb[b], en_hbm.at[pl.ds(0, _C)], sem_out[b]).wait()

    _zero_agg_slice(qb0, agg, sid)
    issue_gidx(0, 0)
    issue_gidx(1, 1)
    issue_q(0, 0)
    issue_q(1, 1)
    plsc.subcore_barrier()
    drain_gidx(0)
    drain_q(0)
    issue_gadds(0)

    def step(g, _):
        for j in range(3):
            b = j
            b1 = (j + 1) % 3
            b2 = (j + 2) % 3
            k = 3 * g + j
            @pl.when(k + 1 < nch)
            def _():
                drain_gidx(b1)
                drain_q(b1)
                issue_gadds(b1)
            @pl.when((k >= 1) & (k <= nch))
            def _():
                drain_out(b2)
            @pl.when(k + 2 < nch)
            def _():
                issue_gidx(k + 2, b2)
                issue_q(k + 2, b2)
            @pl.when(k < nch)
            def _():
                drain_gadds(b)
                _relu_inplace(qb[b])
                base = (start + k) * _C
                pltpu.async_copy(qb[b], en_hbm.at[pl.ds(base, _C)], sem_out[b])
                pltpu.sync_copy(qb[b], agg.at[gis[b]], add=True)
        return _

    lax.fori_loop(0, (ns + 3) // 3, step, None)

    plsc.subcore_barrier()
    pltpu.sync_copy(agg.at[pl.ds(zb, rows_per_tile)],
                    sp_hbm.at[cid, pl.ds(zb, rows_per_tile)])


def _recv_sc_body(en_hbm, r_hbm, rp_hbm, ir0, ir1, buf0, buf1, agg,
                  sem_in0, sem_in1):
    cid = lax.axis_index("c")
    sid = lax.axis_index("s")
    wid = cid * _NSUB + sid
    rows_per_tile = agg.shape[0] // _NSUB
    zb = sid * rows_per_tile

    ir = (ir0, ir1)
    buf = (buf0, buf1)
    sem_in = (sem_in0, sem_in1)

    total_chunks = en_hbm.shape[0] // _C
    start, nch = _tile_chunks(wid, total_chunks)
    ns = total_chunks // (_NCORE * _NSUB) + 1

    def issue_data(k, b):
        base = (start + k) * _C
        pltpu.async_copy(r_hbm.at[pl.ds(base, _C)], ir[b], sem_in[b])
        pltpu.async_copy(en_hbm.at[pl.ds(base, _C)], buf[b], sem_in[b])

    def drain_data(b):
        pltpu.make_async_copy(r_hbm.at[pl.ds(0, _C)], ir[b], sem_in[b]).wait()
        pltpu.make_async_copy(en_hbm.at[pl.ds(0, _C)], buf[b], sem_in[b]).wait()

    _zero_agg_slice(buf0, agg, sid)
    issue_data(0, 0)
    plsc.subcore_barrier()

    def step(g, _):
        for j in range(2):
            b, ob = j, 1 - j
            k = 2 * g + j

            @pl.when(k + 1 < nch)
            def _():
                issue_data(k + 1, ob)

            @pl.when(k < nch)
            def _():
                drain_data(b)
                pltpu.sync_copy(buf[b], agg.at[ir[b]], add=True)
        return _

    lax.fori_loop(0, (ns + 2) // 2, step, None)
    plsc.subcore_barrier()
    pltpu.sync_copy(agg.at[pl.ds(zb, rows_per_tile)],
                    rp_hbm.at[cid, pl.ds(zb, rows_per_tile)])


def kernel(nodes, edges, senders, receivers, globals_, n_node, n_edge,
           W_edge, b_edge, W_node, b_node, W_glob, b_glob):
    N, F = nodes.shape
    E, DE = edges.shape
    DG = globals_.shape[1]
    DEO = b_edge.shape[0]
    DNO = b_node.shape[0]

    senders = senders.astype(jnp.int32)
    receivers = receivers.astype(jnp.int32)

    we_e = W_edge[:DE]
    we_s = W_edge[DE:DE + F]
    we_r = W_edge[DE + F:DE + 2 * F]
    we_g = W_edge[DE + 2 * F:]
    wn_n = W_node[:F]
    wn_s = W_node[F:F + DEO]
    wn_r = W_node[F + DEO:F + 2 * DEO]
    wn_g = W_node[F + 2 * DEO:]
    wg_n = W_glob[:DNO]
    wg_e = W_glob[DNO:DNO + DEO]
    wg_g = W_glob[DNO + DEO:]
    b_edge2 = b_edge.reshape(1, DEO)
    b_node2 = b_node.reshape(1, DNO)
    b_glob2 = b_glob.reshape(1, -1)

    BE = 16000
    q, ps, pr = pl.pallas_call(
        _edge_pre_kernel,
        grid=(E // BE,),
        in_specs=[
            pl.BlockSpec((DE, BE), lambda i: (0, i)),
            pl.BlockSpec((DE, DEO), lambda i: (0, 0)),
            pl.BlockSpec((1, DG), lambda i: (0, 0)),
            pl.BlockSpec((DG, DEO), lambda i: (0, 0)),
            pl.BlockSpec((1, DEO), lambda i: (0, 0)),
            pl.BlockSpec((N, F), lambda i: (0, 0)),
            pl.BlockSpec((F, DEO), lambda i: (0, 0)),
            pl.BlockSpec((F, DEO), lambda i: (0, 0)),
        ],
        out_specs=(pl.BlockSpec((BE, DEO), lambda i: (i, 0)),
                   pl.BlockSpec((N, DEO), lambda i: (0, 0)),
                   pl.BlockSpec((N, DEO), lambda i: (0, 0))),
        out_shape=(jax.ShapeDtypeStruct((E, DEO), F32),
                   jax.ShapeDtypeStruct((N, DEO), F32),
                   jax.ShapeDtypeStruct((N, DEO), F32)),
    )(edges.T, we_e, globals_, we_g, b_edge2, nodes, we_s, we_r)

    NP = _pad_nodes(N)
    mesh = plsc.VectorSubcoreMesh(core_axis_name="c", subcore_axis_name="s")
    edges_new, sent_part = pl.kernel(
        _edge_sc_body,
        out_type=(jax.ShapeDtypeStruct((E, DEO), F32),
                  jax.ShapeDtypeStruct((_NCORE, NP, DEO), F32)),
        mesh=mesh,
        scratch_types=(
            [pltpu.VMEM((_C,), jnp.int32)] * 6
            + [pltpu.VMEM((_C, DEO), F32)] * 3
            + [pltpu.VMEM_SHARED((NP, DEO), F32)]
            + [pltpu.SemaphoreType.DMA] * 12
        ),
    )(ps, pr, q, senders, receivers)

    recv_part = pl.kernel(
        _recv_sc_body,
        out_type=jax.ShapeDtypeStruct((_NCORE, NP, DEO), F32),
        mesh=mesh,
        scratch_types=(
            [pltpu.VMEM((_C,), jnp.int32)] * 2
            + [pltpu.VMEM((_C, DEO), F32)] * 2
            + [pltpu.VMEM_SHARED((NP, DEO), F32)]
            + [pltpu.SemaphoreType.DMA] * 2
        ),
    )(edges_new, receivers)

    BN = 1000
    nodes_new, globals_new = pl.pallas_call(
        _node_glob_kernel,
        grid=(N // BN,),
        in_specs=[
            pl.BlockSpec((BN, F), lambda i: (i, 0)),
            pl.BlockSpec((_NCORE, BN, DEO), lambda i: (0, i, 0)),
            pl.BlockSpec((_NCORE, BN, DEO), lambda i: (0, i, 0)),
            pl.BlockSpec((1, DG), lambda i: (0, 0)),
            pl.BlockSpec((F, DNO), lambda i: (0, 0)),
            pl.BlockSpec((DEO, DNO), lambda i: (0, 0)),
            pl.BlockSpec((DEO, DNO), lambda i: (0, 0)),
            pl.BlockSpec((DG, DNO), lambda i: (0, 0)),
            pl.BlockSpec((1, DNO), lambda i: (0, 0)),
            pl.BlockSpec((DNO, b_glob.shape[0]), lambda i: (0, 0)),
            pl.BlockSpec((DEO, b_glob.shape[0]), lambda i: (0, 0)),
            pl.BlockSpec((DG, b_glob.shape[0]), lambda i: (0, 0)),
            pl.BlockSpec((1, b_glob.shape[0]), lambda i: (0, 0)),
        ],
        out_specs=(pl.BlockSpec((BN, DNO), lambda i: (i, 0)),
                   pl.BlockSpec((1, b_glob.shape[0]), lambda i: (0, 0))),
        out_shape=(jax.ShapeDtypeStruct((N, DNO), F32),
                   jax.ShapeDtypeStruct((1, b_glob.shape[0]), F32)),
        scratch_shapes=[pltpu.VMEM((1, DNO), F32),
                        pltpu.VMEM((1, DEO), F32)],
    )(nodes, sent_part, recv_part, globals_, wn_n, wn_s, wn_r, wn_g,
      b_node2, wg_n, wg_e, wg_g, b_glob2)

    return nodes_new, edges_new, globals_new


# speedup vs baseline: 8.4132x; 1.0034x over previous
import functools

import jax
import jax.numpy as jnp
import numpy as np
from jax import lax
from jax.experimental import pallas as pl
from jax.experimental.pallas import tpu as pltpu
from jax.experimental.pallas import tpu_sc as plsc

F32 = jnp.float32


def _edge_pre_kernel(edges_t_ref, we_ref, g_ref, wg_ref, b_ref, nodes_ref,
                     ws_ref, wr_ref, q_ref, ps_ref, pr_ref):
    ce = jnp.dot(g_ref[...], wg_ref[...], preferred_element_type=F32) + b_ref[...]
    q_ref[...] = lax.dot_general(
        edges_t_ref[...], we_ref[...],
        dimension_numbers=(((0,), (0,)), ((), ())),
        preferred_element_type=F32) + ce

    @pl.when(pl.program_id(0) == 0)
    def _():
        x = nodes_ref[...]
        ps_ref[...] = jnp.dot(x, ws_ref[...], preferred_element_type=F32)
        pr_ref[...] = jnp.dot(x, wr_ref[...], preferred_element_type=F32)


def _node_glob_kernel(nodes_ref, sp_ref, rp_ref, g_ref, wnn_ref, wns_ref,
                      wnr_ref, wng_ref, bn_ref, wgn_ref, wge_ref, wgg_ref,
                      bg_ref, nn_ref, gout_ref, nsum_acc, esum_acc):
    i = pl.program_id(0)
    s_agg = sp_ref[0] + sp_ref[1]
    r_agg = rp_ref[0] + rp_ref[1]
    cn = jnp.dot(g_ref[...], wng_ref[...], preferred_element_type=F32) + bn_ref[...]
    x = (jnp.dot(nodes_ref[...], wnn_ref[...], preferred_element_type=F32)
         + jnp.dot(s_agg, wns_ref[...], preferred_element_type=F32)
         + jnp.dot(r_agg, wnr_ref[...], preferred_element_type=F32)
         + cn)
    nn = jnp.maximum(x, 0.0)
    nn_ref[...] = nn

    @pl.when(i == 0)
    def _():
        nsum_acc[...] = jnp.zeros_like(nsum_acc)
        esum_acc[...] = jnp.zeros_like(esum_acc)

    nsum_acc[...] += jnp.sum(nn, axis=0, keepdims=True)
    esum_acc[...] += jnp.sum(s_agg, axis=0, keepdims=True)

    @pl.when(i == pl.num_programs(0) - 1)
    def _():
        gi = (jnp.dot(nsum_acc[...], wgn_ref[...], preferred_element_type=F32)
              + jnp.dot(esum_acc[...], wge_ref[...], preferred_element_type=F32)
              + jnp.dot(g_ref[...], wgg_ref[...], preferred_element_type=F32)
              + bg_ref[...])
        gout_ref[...] = jnp.maximum(gi, 0.0)


_C = 128
_NSUB = 16
_NCORE = 2


def _pad_nodes(n):
    step = 8 * _NSUB
    return ((n + step - 1) // step) * step


def _zero_vmem_rows(buf, nrows, ncols):
    def row(i, _):
        for j in range(ncols // 16):
            buf[i, pl.ds(j * 16, 16)] = jnp.zeros((16,), F32)
        return _
    lax.fori_loop(0, nrows, row, None)


def _tile_chunks(wid, total_chunks):
    nbase = total_chunks // (_NCORE * _NSUB)
    rem = total_chunks - nbase * _NCORE * _NSUB
    extra = jnp.where(wid < rem, 1, 0)
    start = wid * nbase + jnp.minimum(wid, rem)
    return start, nbase + extra


def _zero_agg_slice(buf, agg, sid):
    rows_per_tile = agg.shape[0] // _NSUB
    zb = sid * rows_per_tile
    _zero_vmem_rows(buf, _C, 128)
    nfull = rows_per_tile // _C
    tail = rows_per_tile - nfull * _C
    for t in range(nfull):
        pltpu.sync_copy(buf, agg.at[pl.ds(zb + t * _C, _C)])
    if tail:
        pltpu.sync_copy(buf.at[pl.ds(0, tail)], agg.at[pl.ds(zb + nfull * _C, tail)])


def _relu_inplace(qb):
    def row(i, _):
        for u in range(2):
            r = 2 * i + u
            for j in range(8):
                sl = pl.ds(j * 16, 16)
                qb[r, sl] = jnp.maximum(qb[r, sl], 0.0)
        return _
    lax.fori_loop(0, _C // 2, row, None)


def _edge_sc_body(ps_hbm, pr_hbm, q_hbm, s_hbm, r_hbm, en_hbm, sp_hbm,
                  gis0, gis1, gis2, gir0, gir1, gir2,
                  qb0, qb1, qb2, agg,
                  sem_q0, sem_q1, sem_q2, sem_in0, sem_in1, sem_in2,
                  sem_out0, sem_out1, sem_out2,
                  sem_gi0, sem_gi1, sem_gi2, sem_sc0, sem_sc1, sem_sc2):
    cid = lax.axis_index("c")
    sid = lax.axis_index("s")
    wid = cid * _NSUB + sid
    rows_per_tile = agg.shape[0] // _NSUB
    zb = sid * rows_per_tile

    gis = (gis0, gis1, gis2)
    gir = (gir0, gir1, gir2)
    qb = (qb0, qb1, qb2)
    sem_q = (sem_q0, sem_q1, sem_q2)
    sem_in = (sem_in0, sem_in1, sem_in2)
    sem_out = (sem_out0, sem_out1, sem_out2)
    sem_gi = (sem_gi0, sem_gi1, sem_gi2)
    sem_sc = (sem_sc0, sem_sc1, sem_sc2)

    total_chunks = q_hbm.shape[0] // _C
    start, nch = _tile_chunks(wid, total_chunks)
    ns = total_chunks // (_NCORE * _NSUB) + 1

    def issue_gidx(k, b):
        base = (start + k) * _C
        pltpu.async_copy(s_hbm.at[pl.ds(base, _C)], gis[b], sem_gi[b])
        pltpu.async_copy(r_hbm.at[pl.ds(base, _C)], gir[b], sem_gi[b])

    def issue_q(k, b):
        base = (start + k) * _C
        pltpu.async_copy(q_hbm.at[pl.ds(base, _C)], qb[b], sem_q[b])

    def issue_gadds(b):
        pltpu.async_copy(ps_hbm.at[gis[b]], qb[b], sem_in[b], add=True)
        pltpu.async_copy(pr_hbm.at[gir[b]], qb[b], sem_in[b], add=True)

    def drain_gidx(b):
        pltpu.make_async_copy(s_hbm.at[pl.ds(0, _C)], gis[b], sem_gi[b]).wait()
        pltpu.make_async_copy(r_hbm.at[pl.ds(0, _C)], gir[b], sem_gi[b]).wait()

    def drain_q(b):
        pltpu.make_async_copy(q_hbm.at[pl.ds(0, _C)], qb[b], sem_q[b]).wait()

    def drain_gadds(b):
        pltpu.make_async_copy(q_hbm.at[pl.ds(0, _C)], qb[b], sem_in[b]).wait()
        pltpu.make_async_copy(q_hbm.at[pl.ds(0, _C)], qb[b], sem_in[b]).wait()

    def drain_out(b):
        pltpu.make_async_copy(qb[b], en_hbm.at[pl.ds(0, _C)], sem_out[b]).wait()
        pltpu.make_async_copy(qb[b], agg.at[gis[b]], sem_sc[b]).wait()

    _zero_agg_slice(qb0, agg, sid)
    issue_gidx(0, 0)
    issue_gidx(1, 1)
    issue_q(0, 0)
    issue_q(1, 1)
    plsc.subcore_barrier()
    drain_gidx(0)
    drain_q(0)
    issue_gadds(0)

    def step(g, _):
        for j in range(3):
            b = j
            b1 = (j + 1) % 3
            b2 = (j + 2) % 3
            k = 3 * g + j
            @pl.when(k + 1 < nch)
            def _():
                drain_gidx(b1)
                drain_q(b1)
                issue_gadds(b1)
            @pl.when((k >= 1) & (k <= nch))
            def _():
                drain_out(b2)
            @pl.when(k + 2 < nch)
            def _():
                issue_gidx(k + 2, b2)
                issue_q(k + 2, b2)
            @pl.when(k < nch)
            def _():
                drain_gadds(b)
                _relu_inplace(qb[b])
                base = (start + k) * _C
                pltpu.async_copy(qb[b], en_hbm.at[pl.ds(base, _C)], sem_out[b])
                pltpu.async_copy(qb[b], agg.at[gis[b]], sem_sc[b], add=True)
        return _

    lax.fori_loop(0, (ns + 3) // 3, step, None)

    plsc.subcore_barrier()
    pltpu.sync_copy(agg.at[pl.ds(zb, rows_per_tile)],
                    sp_hbm.at[cid, pl.ds(zb, rows_per_tile)])


def _recv_sc_body(en_hbm, r_hbm, rp_hbm, ir0, ir1, buf0, buf1, agg,
                  sem_in0, sem_in1, sem_sc0, sem_sc1):
    cid = lax.axis_index("c")
    sid = lax.axis_index("s")
    wid = cid * _NSUB + sid
    rows_per_tile = agg.shape[0] // _NSUB
    zb = sid * rows_per_tile

    ir = (ir0, ir1)
    buf = (buf0, buf1)
    sem_in = (sem_in0, sem_in1)
    sem_sc = (sem_sc0, sem_sc1)

    total_chunks = en_hbm.shape[0] // _C
    start, nch = _tile_chunks(wid, total_chunks)
    ns = total_chunks // (_NCORE * _NSUB) + 1

    def drain_sc(b):
        pltpu.make_async_copy(buf[b], agg.at[ir[b]], sem_sc[b]).wait()

    def issue_data(k, b):
        base = (start + k) * _C
        pltpu.async_copy(r_hbm.at[pl.ds(base, _C)], ir[b], sem_in[b])
        pltpu.async_copy(en_hbm.at[pl.ds(base, _C)], buf[b], sem_in[b])

    def drain_data(b):
        pltpu.make_async_copy(r_hbm.at[pl.ds(0, _C)], ir[b], sem_in[b]).wait()
        pltpu.make_async_copy(en_hbm.at[pl.ds(0, _C)], buf[b], sem_in[b]).wait()

    _zero_agg_slice(buf0, agg, sid)
    issue_data(0, 0)
    plsc.subcore_barrier()

    def step(g, _):
        for j in range(2):
            b, ob = j, 1 - j
            k = 2 * g + j

            @pl.when((k >= 1) & (k <= nch))
            def _():
                drain_sc(ob)

            @pl.when(k + 1 < nch)
            def _():
                issue_data(k + 1, ob)

            @pl.when(k < nch)
            def _():
                drain_data(b)
                pltpu.async_copy(buf[b], agg.at[ir[b]], sem_sc[b], add=True)
        return _

    lax.fori_loop(0, (ns + 2) // 2, step, None)
    plsc.subcore_barrier()
    pltpu.sync_copy(agg.at[pl.ds(zb, rows_per_tile)],
                    rp_hbm.at[cid, pl.ds(zb, rows_per_tile)])


def kernel(nodes, edges, senders, receivers, globals_, n_node, n_edge,
           W_edge, b_edge, W_node, b_node, W_glob, b_glob):
    N, F = nodes.shape
    E, DE = edges.shape
    DG = globals_.shape[1]
    DEO = b_edge.shape[0]
    DNO = b_node.shape[0]

    senders = senders.astype(jnp.int32)
    receivers = receivers.astype(jnp.int32)

    we_e = W_edge[:DE]
    we_s = W_edge[DE:DE + F]
    we_r = W_edge[DE + F:DE + 2 * F]
    we_g = W_edge[DE + 2 * F:]
    wn_n = W_node[:F]
    wn_s = W_node[F:F + DEO]
    wn_r = W_node[F + DEO:F + 2 * DEO]
    wn_g = W_node[F + 2 * DEO:]
    wg_n = W_glob[:DNO]
    wg_e = W_glob[DNO:DNO + DEO]
    wg_g = W_glob[DNO + DEO:]
    b_edge2 = b_edge.reshape(1, DEO)
    b_node2 = b_node.reshape(1, DNO)
    b_glob2 = b_glob.reshape(1, -1)

    BE = 16000
    q, ps, pr = pl.pallas_call(
        _edge_pre_kernel,
        grid=(E // BE,),
        in_specs=[
            pl.BlockSpec((DE, BE), lambda i: (0, i)),
            pl.BlockSpec((DE, DEO), lambda i: (0, 0)),
            pl.BlockSpec((1, DG), lambda i: (0, 0)),
            pl.BlockSpec((DG, DEO), lambda i: (0, 0)),
            pl.BlockSpec((1, DEO), lambda i: (0, 0)),
            pl.BlockSpec((N, F), lambda i: (0, 0)),
            pl.BlockSpec((F, DEO), lambda i: (0, 0)),
            pl.BlockSpec((F, DEO), lambda i: (0, 0)),
        ],
        out_specs=(pl.BlockSpec((BE, DEO), lambda i: (i, 0)),
                   pl.BlockSpec((N, DEO), lambda i: (0, 0)),
                   pl.BlockSpec((N, DEO), lambda i: (0, 0))),
        out_shape=(jax.ShapeDtypeStruct((E, DEO), F32),
                   jax.ShapeDtypeStruct((N, DEO), F32),
                   jax.ShapeDtypeStruct((N, DEO), F32)),
    )(edges.T, we_e, globals_, we_g, b_edge2, nodes, we_s, we_r)

    NP = _pad_nodes(N)
    mesh = plsc.VectorSubcoreMesh(core_axis_name="c", subcore_axis_name="s")
    edges_new, sent_part = pl.kernel(
        _edge_sc_body,
        out_type=(jax.ShapeDtypeStruct((E, DEO), F32),
                  jax.ShapeDtypeStruct((_NCORE, NP, DEO), F32)),
        mesh=mesh,
        scratch_types=(
            [pltpu.VMEM((_C,), jnp.int32)] * 6
            + [pltpu.VMEM((_C, DEO), F32)] * 3
            + [pltpu.VMEM_SHARED((NP, DEO), F32)]
            + [pltpu.SemaphoreType.DMA] * 15
        ),
    )(ps, pr, q, senders, receivers)

    recv_part = pl.kernel(
        _recv_sc_body,
        out_type=jax.ShapeDtypeStruct((_NCORE, NP, DEO), F32),
        mesh=mesh,
        scratch_types=(
            [pltpu.VMEM((_C,), jnp.int32)] * 2
            + [pltpu.VMEM((_C, DEO), F32)] * 2
            + [pltpu.VMEM_SHARED((NP, DEO), F32)]
            + [pltpu.SemaphoreType.DMA] * 4
        ),
    )(edges_new, receivers)

    BN = 1000
    nodes_new, globals_new = pl.pallas_call(
        _node_glob_kernel,
        grid=(N // BN,),
        in_specs=[
            pl.BlockSpec((BN, F), lambda i: (i, 0)),
            pl.BlockSpec((_NCORE, BN, DEO), lambda i: (0, i, 0)),
            pl.BlockSpec((_NCORE, BN, DEO), lambda i: (0, i, 0)),
            pl.BlockSpec((1, DG), lambda i: (0, 0)),
            pl.BlockSpec((F, DNO), lambda i: (0, 0)),
            pl.BlockSpec((DEO, DNO), lambda i: (0, 0)),
            pl.BlockSpec((DEO, DNO), lambda i: (0, 0)),
            pl.BlockSpec((DG, DNO), lambda i: (0, 0)),
            pl.BlockSpec((1, DNO), lambda i: (0, 0)),
            pl.BlockSpec((DNO, b_glob.shape[0]), lambda i: (0, 0)),
            pl.BlockSpec((DEO, b_glob.shape[0]), lambda i: (0, 0)),
            pl.BlockSpec((DG, b_glob.shape[0]), lambda i: (0, 0)),
            pl.BlockSpec((1, b_glob.shape[0]), lambda i: (0, 0)),
        ],
        out_specs=(pl.BlockSpec((BN, DNO), lambda i: (i, 0)),
                   pl.BlockSpec((1, b_glob.shape[0]), lambda i: (0, 0))),
        out_shape=(jax.ShapeDtypeStruct((N, DNO), F32),
                   jax.ShapeDtypeStruct((1, b_glob.shape[0]), F32)),
        scratch_shapes=[pltpu.VMEM((1, DNO), F32),
                        pltpu.VMEM((1, DEO), F32)],
    )(nodes, sent_part, recv_part, globals_, wn_n, wn_s, wn_r, wn_g,
      b_node2, wg_n, wg_e, wg_g, b_glob2)

    return nodes_new, edges_new, globals_new
